# Initial kernel scaffold; baseline (speedup 1.0000x reference)
import jax
import jax.numpy as jnp
from jax.experimental import pallas as pl


def kernel(x, edge_index, Wl1, Wr1, b1, Wl2, Wr2, b2, W3, b3):
    raise NotImplementedError("write your pallas kernel here")


# speedup vs baseline: 4.7658x; 4.7658x over previous
import functools

import jax
import jax.numpy as jnp
from jax import lax
from jax.experimental import pallas as pl
from jax.experimental.pallas import tpu as pltpu
from jax.experimental.pallas import tpu_sc as plsc

N = 10000
NP = 10112
E = 320000
NC = 2
NS = 16
NW = NC * NS
CH = 128
NCHUNK = 80
EP = NW * NCHUNK * CH
RPT = NP // NS


@functools.cache
def _sc_edge_pass(width):
    mesh = plsc.VectorSubcoreMesh(
        core_axis_name="c", subcore_axis_name="s", num_cores=NC, num_subcores=NS
    )

    @functools.partial(
        pl.kernel,
        out_type=jax.ShapeDtypeStruct((NC, NP, width), jnp.float32),
        mesh=mesh,
        scratch_types=[
            pltpu.VMEM((NCHUNK, CH), jnp.int32),
            pltpu.VMEM((NCHUNK, CH), jnp.int32),
            pltpu.VMEM((CH, width), jnp.float32),
            pltpu.VMEM_SHARED((NP, width), jnp.float32),
            pltpu.SemaphoreType.DMA,
        ],
        compiler_params=pltpu.CompilerParams(use_tc_tiling_on_sc=False),
    )
    def sc(rows_hbm, src_hbm, dst_hbm, zeros_hbm, out_hbm,
           src_v, dst_v, buf_v, acc_sp, sem):
        c = lax.axis_index("c")
        s = lax.axis_index("s")
        wid = s * NC + c
        rb = s * RPT
        pltpu.sync_copy(zeros_hbm.at[pl.ds(rb, RPT)], acc_sp.at[pl.ds(rb, RPT)])
        pltpu.sync_copy(src_hbm.at[wid], src_v)
        pltpu.sync_copy(dst_hbm.at[wid], dst_v)
        plsc.subcore_barrier()

        def body(j, carry):
            pltpu.async_copy(rows_hbm.at[src_v.at[j]], buf_v, sem).wait()
            pltpu.sync_copy(buf_v, acc_sp.at[dst_v.at[j]], add=True)
            return carry

        lax.fori_loop(0, NCHUNK, body, 0)
        plsc.subcore_barrier()
        pltpu.sync_copy(acc_sp.at[pl.ds(rb, RPT)],
                        out_hbm.at[c, pl.ds(rb, RPT)])

    return sc


def _tc1_body(x_ref, wl_ref, wr_ref, b1_ref, y1p_ref, xr1_ref):
    xb = x_ref[...]
    y = jnp.dot(xb, wl_ref[...], preferred_element_type=jnp.float32)
    col = lax.broadcasted_iota(jnp.int32, (NP, 80), 1)
    y1p_ref[...] = jnp.where(col == 64, 1.0, y)
    xr1_ref[...] = jnp.dot(xb, wr_ref[...],
                           preferred_element_type=jnp.float32) + b1_ref[...]


def _tc2_body(acc_ref, xr1_ref, wl2_ref, wr2_ref, b2_ref,
              y2_ref, hr2_ref, inv8_ref):
    a = acc_ref[0] + acc_ref[1]
    inv = 1.0 / jnp.maximum(a[:, 64:65], 1.0)
    h = jnp.maximum(a[:, :64] * inv + xr1_ref[...], 0.0)
    y2_ref[...] = jnp.dot(h, wl2_ref[...], preferred_element_type=jnp.float32)
    hr2_ref[...] = jnp.dot(h, wr2_ref[...],
                           preferred_element_type=jnp.float32) + b2_ref[...]
    inv8_ref[...] = jnp.broadcast_to(inv, (NP, 8))


def _tc3_body(acc2_ref, hr2_ref, inv8_ref, w3_ref, b3_ref, out_ref):
    a2 = acc2_ref[0] + acc2_ref[1]
    h2 = a2 * inv8_ref[...][:, 0:1] + hr2_ref[...]
    out_ref[...] = jnp.dot(h2, w3_ref[...],
                           preferred_element_type=jnp.float32) + b3_ref[...]


def kernel(x, edge_index, Wl1, Wr1, b1, Wl2, Wr2, b2, W3, b3):
    xpad = jnp.pad(x, ((0, NP - N), (0, 0)))
    ei = edge_index.astype(jnp.int32)
    pad = jnp.full((EP - E,), N, jnp.int32)
    srcr = jnp.concatenate([ei[0], pad]).reshape(NW, NCHUNK, CH)
    dstr = jnp.concatenate([ei[1], pad]).reshape(NW, NCHUNK, CH)
    wl1p = jnp.pad(Wl1, ((0, 0), (0, 16)))
    z80 = jnp.zeros((NP, 80), jnp.float32)
    z32 = jnp.zeros((NP, 32), jnp.float32)

    y1p, xr1 = pl.pallas_call(
        _tc1_body,
        out_shape=(jax.ShapeDtypeStruct((NP, 80), jnp.float32),
                   jax.ShapeDtypeStruct((NP, 64), jnp.float32)),
    )(xpad, wl1p, Wr1, b1.reshape(1, 64))

    acc1 = _sc_edge_pass(80)(y1p, srcr, dstr, z80)

    y2p, hr2, inv8 = pl.pallas_call(
        _tc2_body,
        out_shape=(jax.ShapeDtypeStruct((NP, 32), jnp.float32),
                   jax.ShapeDtypeStruct((NP, 32), jnp.float32),
                   jax.ShapeDtypeStruct((NP, 8), jnp.float32)),
    )(acc1, xr1, Wl2, Wr2, b2.reshape(1, 32))

    acc2 = _sc_edge_pass(32)(y2p, srcr, dstr, z32)

    w3p = jnp.pad(W3, ((0, 0), (0, 6)))
    b3p = jnp.pad(b3, (0, 6)).reshape(1, 8)
    outp = pl.pallas_call(
        _tc3_body,
        out_shape=jax.ShapeDtypeStruct((NP, 8), jnp.float32),
    )(acc2, hr2, inv8, w3p, b3p)
    return outp[:N, :2]


# speedup vs baseline: 5.6872x; 1.1933x over previous
import functools

import jax
import jax.numpy as jnp
from jax import lax
from jax.experimental import pallas as pl
from jax.experimental.pallas import tpu as pltpu
from jax.experimental.pallas import tpu_sc as plsc

N = 10000
NP = 10112
E = 320000
NC = 2
NS = 16
NW = NC * NS
CH = 128
NCHUNK = 80
EP = NW * NCHUNK * CH
RPT = NP // NS
NBUF = 4


@functools.cache
def _sc_edge_pass(width):
    mesh = plsc.VectorSubcoreMesh(
        core_axis_name="c", subcore_axis_name="s", num_cores=NC, num_subcores=NS
    )

    @functools.partial(
        pl.kernel,
        out_type=jax.ShapeDtypeStruct((NC, NP, width), jnp.float32),
        mesh=mesh,
        scratch_types=[
            pltpu.VMEM((NCHUNK, CH), jnp.int32),
            pltpu.VMEM((NCHUNK, CH), jnp.int32),
            pltpu.VMEM((NBUF, CH, width), jnp.float32),
            pltpu.VMEM_SHARED((NP, width), jnp.float32),
            [pltpu.SemaphoreType.DMA] * NBUF,
        ],
        compiler_params=pltpu.CompilerParams(use_tc_tiling_on_sc=False),
    )
    def sc(rows_hbm, src_hbm, dst_hbm, zeros_hbm, out_hbm,
           src_v, dst_v, buf_v, acc_sp, sems):
        c = lax.axis_index("c")
        s = lax.axis_index("s")
        wid = s * NC + c
        rb = s * RPT
        pltpu.sync_copy(zeros_hbm.at[pl.ds(rb, RPT)], acc_sp.at[pl.ds(rb, RPT)])
        pltpu.sync_copy(src_hbm.at[wid], src_v)
        pltpu.sync_copy(dst_hbm.at[wid], dst_v)
        plsc.subcore_barrier()

        def fire(j, b):
            pltpu.async_copy(rows_hbm.at[src_v.at[j]], buf_v.at[b], sems[b])

        for b in range(NBUF):
            fire(b, b)

        def group(g, carry):
            for b in range(NBUF):
                j = g * NBUF + b
                pltpu.make_async_copy(rows_hbm.at[src_v.at[j]],
                                      buf_v.at[b], sems[b]).wait()
                pltpu.sync_copy(buf_v.at[b], acc_sp.at[dst_v.at[j]], add=True)
                fire(jnp.minimum(j + NBUF, NCHUNK - 1), b)
            return carry

        lax.fori_loop(0, NCHUNK // NBUF, group, 0)
        for b in range(NBUF):
            pltpu.make_async_copy(rows_hbm.at[src_v.at[NCHUNK - 1]],
                                  buf_v.at[b], sems[b]).wait()
        plsc.subcore_barrier()
        pltpu.sync_copy(acc_sp.at[pl.ds(rb, RPT)],
                        out_hbm.at[c, pl.ds(rb, RPT)])

    return sc


def _tc1_body(x_ref, wl_ref, wr_ref, b1_ref, y1p_ref, xr1_ref):
    xb = x_ref[...]
    y = jnp.dot(xb, wl_ref[...], preferred_element_type=jnp.float32)
    col = lax.broadcasted_iota(jnp.int32, (NP, 80), 1)
    y1p_ref[...] = jnp.where(col == 64, 1.0, y)
    xr1_ref[...] = jnp.dot(xb, wr_ref[...],
                           preferred_element_type=jnp.float32) + b1_ref[...]


def _tc2_body(acc_ref, xr1_ref, wl2_ref, wr2_ref, b2_ref,
              y2_ref, hr2_ref, inv8_ref):
    a = acc_ref[0] + acc_ref[1]
    inv = 1.0 / jnp.maximum(a[:, 64:65], 1.0)
    h = jnp.maximum(a[:, :64] * inv + xr1_ref[...], 0.0)
    y2_ref[...] = jnp.dot(h, wl2_ref[...], preferred_element_type=jnp.float32)
    hr2_ref[...] = jnp.dot(h, wr2_ref[...],
                           preferred_element_type=jnp.float32) + b2_ref[...]
    inv8_ref[...] = jnp.broadcast_to(inv, (NP, 8))


def _tc3_body(acc2_ref, hr2_ref, inv8_ref, w3_ref, b3_ref, out_ref):
    a2 = acc2_ref[0] + acc2_ref[1]
    h2 = a2 * inv8_ref[...][:, 0:1] + hr2_ref[...]
    out_ref[...] = jnp.dot(h2, w3_ref[...],
                           preferred_element_type=jnp.float32) + b3_ref[...]


def kernel(x, edge_index, Wl1, Wr1, b1, Wl2, Wr2, b2, W3, b3):
    xpad = jnp.pad(x, ((0, NP - N), (0, 0)))
    ei = edge_index.astype(jnp.int32)
    pad = jnp.full((EP - E,), N, jnp.int32)
    srcr = jnp.concatenate([ei[0], pad]).reshape(NW, NCHUNK, CH)
    dstr = jnp.concatenate([ei[1], pad]).reshape(NW, NCHUNK, CH)
    wl1p = jnp.pad(Wl1, ((0, 0), (0, 16)))
    z80 = jnp.zeros((NP, 80), jnp.float32)
    z32 = jnp.zeros((NP, 32), jnp.float32)

    y1p, xr1 = pl.pallas_call(
        _tc1_body,
        out_shape=(jax.ShapeDtypeStruct((NP, 80), jnp.float32),
                   jax.ShapeDtypeStruct((NP, 64), jnp.float32)),
    )(xpad, wl1p, Wr1, b1.reshape(1, 64))

    acc1 = _sc_edge_pass(80)(y1p, srcr, dstr, z80)

    y2p, hr2, inv8 = pl.pallas_call(
        _tc2_body,
        out_shape=(jax.ShapeDtypeStruct((NP, 32), jnp.float32),
                   jax.ShapeDtypeStruct((NP, 32), jnp.float32),
                   jax.ShapeDtypeStruct((NP, 8), jnp.float32)),
    )(acc1, xr1, Wl2, Wr2, b2.reshape(1, 32))

    acc2 = _sc_edge_pass(32)(y2p, srcr, dstr, z32)

    w3p = jnp.pad(W3, ((0, 0), (0, 6)))
    b3p = jnp.pad(b3, (0, 6)).reshape(1, 8)
    outp = pl.pallas_call(
        _tc3_body,
        out_shape=jax.ShapeDtypeStruct((NP, 8), jnp.float32),
    )(acc2, hr2, inv8, w3p, b3p)
    return outp[:N, :2]
